# baseline (speedup 1.0000x reference)
import functools

import jax
import jax.numpy as jnp
import numpy as np
from jax import lax
from jax.experimental import pallas as pl
from jax.experimental.pallas import tpu as pltpu
from jax.experimental.pallas import tpu_sc as plsc

VOCAB = 100000
MAX_SEQ_LEN = 2048
D_MODEL = 768

B = 4
S = 2048
NW = 32
S_PER_W = S // NW
LANES = 16
VECS_PER_ROW = D_MODEL // LANES

CH = 8
NGRP = S_PER_W // CH
GR = B * CH
RING = 3


def _pos_encoding(max_seq_len, d_model):
    pos = np.arange(max_seq_len, dtype=np.float32)[:, None]
    div = np.exp(
        np.arange(0, d_model, 2, dtype=np.float32) * (-np.log(10000.0) / d_model)
    )
    pe = np.zeros((max_seq_len, d_model), dtype=np.float32)
    pe[:, 0::2] = np.sin(pos * div)
    pe[:, 1::2] = np.cos(pos * div)
    return pe


_PE = _pos_encoding(MAX_SEQ_LEN, D_MODEL)


def _make_sc_call():
    mesh = plsc.VectorSubcoreMesh(core_axis_name="c", subcore_axis_name="s")

    @functools.partial(
        pl.kernel,
        mesh=mesh,
        out_type=jax.ShapeDtypeStruct((B, S, D_MODEL), jnp.float32),
        scratch_types=[
            pltpu.VMEM((NGRP, GR), jnp.int32),
            pltpu.VMEM((S_PER_W, D_MODEL), jnp.float32),
        ]
        + [pltpu.VMEM((GR, D_MODEL), jnp.float32) for _ in range(RING)]
        + [pltpu.SemaphoreType.DMA for _ in range(2 * RING + 1)],
    )
    def emb_kernel(xt_hbm, w_hbm, pe_hbm, out_hbm, idx_v, pe_v, *bufs_and_sems):
        rows = bufs_and_sems[:RING]
        gsem = bufs_and_sems[RING:2 * RING]
        osem = bufs_and_sems[2 * RING:3 * RING]
        psem = bufs_and_sems[3 * RING]
        wid = lax.axis_index("s") * 2 + lax.axis_index("c")
        seq_base = wid * S_PER_W

        pe_copy = pltpu.async_copy(pe_hbm.at[wid], pe_v, psem)
        pltpu.sync_copy(xt_hbm.at[wid], idx_v)

        def fire_gather(h):
            s = h % RING
            return pltpu.async_copy(w_hbm.at[idx_v.at[h]], rows[s], gsem[s])

        def fire_outs(h):
            s = h % RING
            return [
                pltpu.async_copy(
                    rows[s].at[pl.ds(b * CH, CH)],
                    out_hbm.at[b, pl.ds(seq_base + h * CH, CH)], osem[s])
                for b in range(B)
            ]

        gathers = {}
        outs = {}
        for h in range(min(RING - 1, NGRP)):
            gathers[h] = fire_gather(h)
        pe_copy.wait()

        for h in range(NGRP):
            nh = h + RING - 1
            if nh < NGRP:
                if nh >= RING:
                    for d in outs[nh - RING]:
                        d.wait()
                gathers[nh] = fire_gather(nh)
            gathers[h].wait()

            s = h % RING

            def add_row(r, _, _h=h, _s=s):
                for j in range(VECS_PER_ROW):
                    sl = pl.ds(j * LANES, LANES)
                    pv = pe_v[_h * CH + r, sl]
                    for b in range(B):
                        plsc.addupdate(rows[_s].at[b * CH + r, sl], pv)
                return 0

            lax.fori_loop(0, CH, add_row, 0)
            outs[h] = fire_outs(h)

        for h in range(max(0, NGRP - RING), NGRP):
            for d in outs[h]:
                d.wait()

    return emb_kernel


_SC_CALL = _make_sc_call()


def kernel(x, W):
    xt = (x.astype(jnp.int32)
          .reshape(B, NW, NGRP, CH)
          .transpose(1, 2, 0, 3)
          .reshape(NW, NGRP, B * CH))
    pe = jnp.asarray(_PE).reshape(NW, S_PER_W, D_MODEL)
    return _SC_CALL(xt, W, pe)
